# baseline (speedup 1.0000x reference)
import functools

import jax
import jax.numpy as jnp
from jax.experimental import pallas as pl
from jax.experimental.pallas import tpu as pltpu

B, I, H, U = 4096, 128, 512, 512
TB = 512


def _cfc_kernel(inp_ref, hx_ref, ts_ref, wbt_ref, wbb_ref, bb_ref,
                wh_ref, bh_ref, out_ref):
    x = jnp.tanh(
        jnp.dot(inp_ref[...], wbt_ref[...], preferred_element_type=jnp.float32)
        + jnp.dot(hx_ref[...], wbb_ref[...], preferred_element_type=jnp.float32)
        + bb_ref[...]
    )
    h4 = jnp.dot(x.astype(jnp.bfloat16), wh_ref[...],
                 preferred_element_type=jnp.float32) + bh_ref[...]
    ff1 = jnp.tanh(h4[:, :H])
    ff2 = jnp.tanh(h4[:, H:2 * H])
    t_a = h4[:, 2 * H:3 * H]
    t_b = h4[:, 3 * H:]
    s = jax.nn.sigmoid(t_a * ts_ref[...] + t_b)
    out_ref[...] = ff1 * (1.0 - s) + s * ff2


@functools.partial(jax.jit, static_argnames=())
def kernel(input, hx, ts, Wb, bb, W_ff1, b_ff1, W_ff2, b_ff2, W_ta, b_ta, W_tb, b_tb):
    bf16 = jnp.bfloat16
    Wh = jnp.concatenate([W_ff1, W_ff2, W_ta, W_tb], axis=1).astype(bf16)
    bh = jnp.concatenate([b_ff1, b_ff2, b_ta, b_tb])[None, :]
    Wb_top = Wb[:I].astype(bf16)
    Wb_bot = Wb[I:].astype(bf16)
    bb2 = bb[None, :]
    ts2 = ts[:, None]
    input = input.astype(bf16)
    hx = hx.astype(bf16)

    grid = (B // TB,)
    out = pl.pallas_call(
        _cfc_kernel,
        grid=grid,
        in_specs=[
            pl.BlockSpec((TB, I), lambda i: (i, 0)),
            pl.BlockSpec((TB, H), lambda i: (i, 0)),
            pl.BlockSpec((TB, 1), lambda i: (i, 0)),
            pl.BlockSpec((I, U), lambda i: (0, 0)),
            pl.BlockSpec((H, U), lambda i: (0, 0)),
            pl.BlockSpec((1, U), lambda i: (0, 0)),
            pl.BlockSpec((U, 4 * H), lambda i: (0, 0)),
            pl.BlockSpec((1, 4 * H), lambda i: (0, 0)),
        ],
        out_specs=pl.BlockSpec((TB, H), lambda i: (i, 0)),
        out_shape=jax.ShapeDtypeStruct((B, H), jnp.float32),
        compiler_params=pltpu.CompilerParams(
            dimension_semantics=("arbitrary",),
        ),
    )(input, hx, ts2, Wb_top, Wb_bot, bb2, Wh, bh)
    return (out, out)


# speedup vs baseline: 1.4470x; 1.4470x over previous
import jax
import jax.numpy as jnp
from jax.experimental import pallas as pl
from jax.experimental.pallas import tpu as pltpu

B, I, H, U = 4096, 128, 512, 512
TB = 512


def _bf(a):
    return a.astype(jnp.bfloat16)


def _cfc_kernel(inp_ref, hx_ref, ts_ref, wb_ref, bb_ref,
                w1_ref, b1_ref, w2_ref, b2_ref,
                wa_ref, ba_ref, wtb_ref, btb_ref, out_ref):
    wb = _bf(wb_ref[...])
    x = jnp.tanh(
        jnp.dot(_bf(inp_ref[...]), wb[:I], preferred_element_type=jnp.float32)
        + jnp.dot(_bf(hx_ref[...]), wb[I:], preferred_element_type=jnp.float32)
        + bb_ref[...]
    )
    xb = _bf(x)
    ff1 = jnp.tanh(jnp.dot(xb, _bf(w1_ref[...]),
                           preferred_element_type=jnp.float32) + b1_ref[...])
    ff2 = jnp.tanh(jnp.dot(xb, _bf(w2_ref[...]),
                           preferred_element_type=jnp.float32) + b2_ref[...])
    t_a = jnp.dot(xb, _bf(wa_ref[...]),
                  preferred_element_type=jnp.float32) + ba_ref[...]
    t_b = jnp.dot(xb, _bf(wtb_ref[...]),
                  preferred_element_type=jnp.float32) + btb_ref[...]
    s = jax.nn.sigmoid(t_a * ts_ref[...] + t_b)
    out_ref[...] = ff1 * (1.0 - s) + s * ff2


def kernel(input, hx, ts, Wb, bb, W_ff1, b_ff1, W_ff2, b_ff2, W_ta, b_ta, W_tb, b_tb):
    ts2 = ts[:, None]
    bb2 = bb[None, :]
    b1 = b_ff1[None, :]
    b2 = b_ff2[None, :]
    ba = b_ta[None, :]
    btb = b_tb[None, :]

    whole = lambda shape: pl.BlockSpec(shape, lambda i: (0,) * len(shape))
    out = pl.pallas_call(
        _cfc_kernel,
        grid=(B // TB,),
        in_specs=[
            pl.BlockSpec((TB, I), lambda i: (i, 0)),
            pl.BlockSpec((TB, H), lambda i: (i, 0)),
            pl.BlockSpec((TB, 1), lambda i: (i, 0)),
            whole((I + H, U)),
            whole((1, U)),
            whole((U, H)), whole((1, H)),
            whole((U, H)), whole((1, H)),
            whole((U, H)), whole((1, H)),
            whole((U, H)), whole((1, H)),
        ],
        out_specs=pl.BlockSpec((TB, H), lambda i: (i, 0)),
        out_shape=jax.ShapeDtypeStruct((B, H), jnp.float32),
        compiler_params=pltpu.CompilerParams(
            dimension_semantics=("arbitrary",),
        ),
    )(input, hx, ts2, Wb, bb2, W_ff1, b1, W_ff2, b2, W_ta, ba, W_tb, btb)
    return (out, out)


# speedup vs baseline: 1.4511x; 1.0028x over previous
import jax
import jax.numpy as jnp
from jax.experimental import pallas as pl
from jax.experimental.pallas import tpu as pltpu

B, I, H, U = 4096, 128, 512, 512
TB = 512


def _bf(a):
    return a.astype(jnp.bfloat16)


def _cfc_kernel(inp_ref, hx_ref, ts_ref, wb_ref, bb_ref,
                w1_ref, b1_ref, w2_ref, b2_ref,
                wa_ref, ba_ref, wtb_ref, btb_ref, out_ref,
                wb_s, w1_s, w2_s, wa_s, wtb_s):
    @pl.when(pl.program_id(0) == 0)
    def _():
        wb_s[...] = _bf(wb_ref[...])
        w1_s[...] = _bf(w1_ref[...])
        w2_s[...] = _bf(w2_ref[...])
        wa_s[...] = _bf(wa_ref[...])
        wtb_s[...] = _bf(wtb_ref[...])

    x = jnp.tanh(
        jnp.dot(_bf(inp_ref[...]), wb_s[:I], preferred_element_type=jnp.float32)
        + jnp.dot(_bf(hx_ref[...]), wb_s[I:], preferred_element_type=jnp.float32)
        + bb_ref[...]
    )
    xb = _bf(x)
    ff1 = jnp.tanh(jnp.dot(xb, w1_s[...],
                           preferred_element_type=jnp.float32) + b1_ref[...])
    ff2 = jnp.tanh(jnp.dot(xb, w2_s[...],
                           preferred_element_type=jnp.float32) + b2_ref[...])
    t_a = jnp.dot(xb, wa_s[...],
                  preferred_element_type=jnp.float32) + ba_ref[...]
    t_b = jnp.dot(xb, wtb_s[...],
                  preferred_element_type=jnp.float32) + btb_ref[...]
    s = jax.nn.sigmoid(t_a * ts_ref[...] + t_b)
    out_ref[...] = ff1 + s * (ff2 - ff1)


def kernel(input, hx, ts, Wb, bb, W_ff1, b_ff1, W_ff2, b_ff2, W_ta, b_ta, W_tb, b_tb):
    ts2 = ts[:, None]
    bb2 = bb[None, :]
    b1 = b_ff1[None, :]
    b2 = b_ff2[None, :]
    ba = b_ta[None, :]
    btb = b_tb[None, :]

    whole = lambda shape: pl.BlockSpec(shape, lambda i: (0,) * len(shape))
    bf16 = jnp.bfloat16
    out = pl.pallas_call(
        _cfc_kernel,
        grid=(B // TB,),
        in_specs=[
            pl.BlockSpec((TB, I), lambda i: (i, 0)),
            pl.BlockSpec((TB, H), lambda i: (i, 0)),
            pl.BlockSpec((TB, 1), lambda i: (i, 0)),
            whole((I + H, U)),
            whole((1, U)),
            whole((U, H)), whole((1, H)),
            whole((U, H)), whole((1, H)),
            whole((U, H)), whole((1, H)),
            whole((U, H)), whole((1, H)),
        ],
        out_specs=pl.BlockSpec((TB, H), lambda i: (i, 0)),
        out_shape=jax.ShapeDtypeStruct((B, H), jnp.float32),
        scratch_shapes=[
            pltpu.VMEM((I + H, U), bf16),
            pltpu.VMEM((U, H), bf16),
            pltpu.VMEM((U, H), bf16),
            pltpu.VMEM((U, H), bf16),
            pltpu.VMEM((U, H), bf16),
        ],
        compiler_params=pltpu.CompilerParams(
            dimension_semantics=("arbitrary",),
        ),
    )(input, hx, ts2, Wb, bb2, W_ff1, b1, W_ff2, b2, W_ta, ba, W_tb, btb)
    return (out, out)


# speedup vs baseline: 1.4630x; 1.0081x over previous
import jax
import jax.numpy as jnp
from jax.experimental import pallas as pl
from jax.experimental.pallas import tpu as pltpu

B, I, H, U = 4096, 128, 512, 512
TB = 512


def _bf(a):
    return a.astype(jnp.bfloat16)


def _cfc_kernel(inp_ref, hx_ref, ts_ref, wb_ref, bb_ref,
                w1_ref, b1_ref, w2_ref, b2_ref,
                wa_ref, ba_ref, wtb_ref, btb_ref, out_ref):
    wb = _bf(wb_ref[...])
    x = jnp.tanh(
        jnp.dot(_bf(inp_ref[...]), wb[:I], preferred_element_type=jnp.float32)
        + jnp.dot(_bf(hx_ref[...]), wb[I:], preferred_element_type=jnp.float32)
        + bb_ref[...]
    )
    xb = _bf(x)
    ff1 = jnp.tanh(jnp.dot(xb, _bf(w1_ref[...]),
                           preferred_element_type=jnp.float32) + b1_ref[...])
    ff2 = jnp.tanh(jnp.dot(xb, _bf(w2_ref[...]),
                           preferred_element_type=jnp.float32) + b2_ref[...])
    t_a = jnp.dot(xb, _bf(wa_ref[...]),
                  preferred_element_type=jnp.float32) + ba_ref[...]
    t_b = jnp.dot(xb, _bf(wtb_ref[...]),
                  preferred_element_type=jnp.float32) + btb_ref[...]
    s = jax.nn.sigmoid(t_a * ts_ref[...] + t_b)
    out_ref[...] = ff1 + s * (ff2 - ff1)


def kernel(input, hx, ts, Wb, bb, W_ff1, b_ff1, W_ff2, b_ff2, W_ta, b_ta, W_tb, b_tb):
    ts2 = ts[:, None]
    bb2 = bb[None, :]
    b1 = b_ff1[None, :]
    b2 = b_ff2[None, :]
    ba = b_ta[None, :]
    btb = b_tb[None, :]

    whole = lambda shape: pl.BlockSpec(shape, lambda i: (0,) * len(shape))
    out = pl.pallas_call(
        _cfc_kernel,
        grid=(B // TB,),
        in_specs=[
            pl.BlockSpec((TB, I), lambda i: (i, 0)),
            pl.BlockSpec((TB, H), lambda i: (i, 0)),
            pl.BlockSpec((TB, 1), lambda i: (i, 0)),
            whole((I + H, U)),
            whole((1, U)),
            whole((U, H)), whole((1, H)),
            whole((U, H)), whole((1, H)),
            whole((U, H)), whole((1, H)),
            whole((U, H)), whole((1, H)),
        ],
        out_specs=pl.BlockSpec((TB, H), lambda i: (i, 0)),
        out_shape=jax.ShapeDtypeStruct((B, H), jnp.float32),
        compiler_params=pltpu.CompilerParams(
            dimension_semantics=("parallel",),
        ),
    )(input, hx, ts2, Wb, bb2, W_ff1, b1, W_ff2, b2, W_ta, ba, W_tb, btb)
    return (out, out)


# speedup vs baseline: 1.8649x; 1.2747x over previous
import jax
import jax.numpy as jnp
from jax.experimental import pallas as pl
from jax.experimental.pallas import tpu as pltpu

B, I, H, U = 4096, 128, 512, 512
TB = 512


def _bf(a):
    return a.astype(jnp.bfloat16)


def _cfc_kernel(inp_ref, hx_ref, ts_ref, wb_ref, bb_ref,
                w1_ref, b1_ref, w2_ref, b2_ref,
                wa_ref, ba_ref, wtb_ref, btb_ref, out_ref, out2_ref):
    wb = _bf(wb_ref[...])
    x = jnp.tanh(
        jnp.dot(_bf(inp_ref[...]), wb[:I], preferred_element_type=jnp.float32)
        + jnp.dot(_bf(hx_ref[...]), wb[I:], preferred_element_type=jnp.float32)
        + bb_ref[...]
    )
    xb = _bf(x)
    ff1 = jnp.tanh(jnp.dot(xb, _bf(w1_ref[...]),
                           preferred_element_type=jnp.float32) + b1_ref[...])
    ff2 = jnp.tanh(jnp.dot(xb, _bf(w2_ref[...]),
                           preferred_element_type=jnp.float32) + b2_ref[...])
    t_a = jnp.dot(xb, _bf(wa_ref[...]),
                  preferred_element_type=jnp.float32) + ba_ref[...]
    t_b = jnp.dot(xb, _bf(wtb_ref[...]),
                  preferred_element_type=jnp.float32) + btb_ref[...]
    s = jax.nn.sigmoid(t_a * ts_ref[...] + t_b)
    res = ff1 + s * (ff2 - ff1)
    out_ref[...] = res
    out2_ref[...] = res


def kernel(input, hx, ts, Wb, bb, W_ff1, b_ff1, W_ff2, b_ff2, W_ta, b_ta, W_tb, b_tb):
    ts2 = ts[:, None]
    bb2 = bb[None, :]
    b1 = b_ff1[None, :]
    b2 = b_ff2[None, :]
    ba = b_ta[None, :]
    btb = b_tb[None, :]

    whole = lambda shape: pl.BlockSpec(shape, lambda i: (0,) * len(shape))
    out = pl.pallas_call(
        _cfc_kernel,
        grid=(B // TB,),
        in_specs=[
            pl.BlockSpec((TB, I), lambda i: (i, 0)),
            pl.BlockSpec((TB, H), lambda i: (i, 0)),
            pl.BlockSpec((TB, 1), lambda i: (i, 0)),
            whole((I + H, U)),
            whole((1, U)),
            whole((U, H)), whole((1, H)),
            whole((U, H)), whole((1, H)),
            whole((U, H)), whole((1, H)),
            whole((U, H)), whole((1, H)),
        ],
        out_specs=[pl.BlockSpec((TB, H), lambda i: (i, 0)),
                   pl.BlockSpec((TB, H), lambda i: (i, 0))],
        out_shape=[jax.ShapeDtypeStruct((B, H), jnp.float32),
                   jax.ShapeDtypeStruct((B, H), jnp.float32)],
        compiler_params=pltpu.CompilerParams(
            dimension_semantics=("parallel",),
        ),
    )(input, hx, ts2, Wb, bb2, W_ff1, b1, W_ff2, b2, W_ta, ba, W_tb, btb)
    return (out[0], out[1])
